# baseline (speedup 1.0000x reference)
import dataclasses
import functools

import jax
import jax.numpy as jnp
import numpy as np
from jax import lax
from jax.experimental import pallas as pl
from jax.experimental.pallas import tpu as pltpu
from jax.experimental.pallas import tpu_sc as plsc

N = 10000
D = 128
NUM_LAYERS = 3
ALPHA = 0.5

NC = 2
NS = 16
NT = NC * NS
L = 16

NP = 10240
NACC = 10112
CHUNK = 128
NCHUNK = 162
GB = 9
NBUF = 3
SLAB = NCHUNK * CHUNK
E_PAD = NS * SLAB

ROWS_PER_TILE = NACC // NS

_ISIG = np.empty((D,), np.int32)
for _k in range(D // 32):
    for _j in range(16):
        _ISIG[32 * _k + 2 * _j] = 32 * _k + _j
        _ISIG[32 * _k + 2 * _j + 1] = 32 * _k + 16 + _j

_mesh = plsc.VectorSubcoreMesh(
    core_axis_name="c", subcore_axis_name="s", num_cores=NC, num_subcores=NS
)

_sc_params = pltpu.CompilerParams()
if "needs_layout_passes" in pltpu.CompilerParams.__dataclass_fields__:
    _sc_params = dataclasses.replace(_sc_params, needs_layout_passes=False)
if "use_tc_tiling_on_sc" in pltpu.CompilerParams.__dataclass_fields__:
    _sc_params = dataclasses.replace(_sc_params, use_tc_tiling_on_sc=False)


@functools.partial(
    pl.kernel,
    out_type=jax.ShapeDtypeStruct((NT, 2, NP), jnp.float32),
    mesh=_mesh,
    scratch_types=[
        pltpu.VMEM((NCHUNK, CHUNK), jnp.int32),
        pltpu.VMEM((NCHUNK, CHUNK), jnp.int32),
        pltpu.VMEM((NCHUNK, CHUNK), jnp.float32),
        pltpu.VMEM((NP,), jnp.float32),
        pltpu.VMEM((NP,), jnp.float32),
    ],
    compiler_params=_sc_params,
)
def _deg_kernel(src_hbm, dst_hbm, ew_hbm, part_hbm, src_v, dst_v, ew_v,
                acco_v, acci_v):
    c = lax.axis_index("c")
    s = lax.axis_index("s")
    pltpu.sync_copy(src_hbm.at[s], src_v)
    pltpu.sync_copy(dst_hbm.at[s], dst_v)
    pltpu.sync_copy(ew_hbm.at[s], ew_v)

    zero = jnp.zeros((L,), jnp.float32)

    @pl.loop(0, NP // L)
    def _(i):
        acco_v.at[pl.ds(i * L, L)][...] = zero
        acci_v.at[pl.ds(i * L, L)][...] = zero

    half = NCHUNK // 2

    @pl.loop(0, half)
    def _(jj):
        j = c * half + jj

        @pl.loop(0, CHUNK // L)
        def _(g):
            sv = src_v.at[j, pl.ds(g * L, L)][...]
            dv = dst_v.at[j, pl.ds(g * L, L)][...]
            wv = ew_v.at[j, pl.ds(g * L, L)][...]
            plsc.addupdate_scatter(acco_v, [sv], wv)
            plsc.addupdate_scatter(acci_v, [dv], wv)

    w = c * NS + s
    pltpu.sync_copy(acco_v, part_hbm.at[w, 0])
    pltpu.sync_copy(acci_v, part_hbm.at[w, 1])


@functools.partial(
    pl.kernel,
    out_type=(
        jax.ShapeDtypeStruct((NP, D), jnp.float32),
        jax.ShapeDtypeStruct((NP, D), jnp.float32),
    ),
    mesh=_mesh,
    scratch_types=[
        pltpu.VMEM((GB, CHUNK), jnp.int32),
        pltpu.VMEM((GB, CHUNK), jnp.int32),
        pltpu.VMEM((GB, CHUNK), jnp.float32),
        pltpu.VMEM((CHUNK, D), jnp.bfloat16),
        pltpu.VMEM((CHUNK, D), jnp.bfloat16),
        pltpu.VMEM((CHUNK, D), jnp.bfloat16),
        pltpu.VMEM((CHUNK, D), jnp.float32),
        pltpu.VMEM_SHARED((NACC, D), jnp.float32),
        pltpu.SemaphoreType.DMA,
        pltpu.SemaphoreType.DMA,
        pltpu.SemaphoreType.DMA,
        pltpu.SemaphoreType.DMA,
    ],
    compiler_params=_sc_params,
)
def _edge_kernel(u_hbm, v_hbm, src_hbm, dst_hbm, ew_hbm, seg1_hbm, seg2_hbm,
                 gidx_v, sidx_v, ew_v, in_a, in_b, in_c, out_v, acc_sh,
                 gsem_a, gsem_b, gsem_c, ssem):
    c = lax.axis_index("c")
    s = lax.axis_index("s")
    ins = (in_a, in_b, in_c)
    gsems = (gsem_a, gsem_b, gsem_c)
    zero = jnp.zeros((L,), jnp.float32)
    hmask = jnp.int32(-65536)

    def start_gather(buf, j):
        idx = gidx_v.at[j]

        @pl.when(c == 0)
        def _():
            pltpu.async_copy(u_hbm.at[idx], ins[buf], gsems[buf])

        @pl.when(c != 0)
        def _():
            pltpu.async_copy(v_hbm.at[idx], ins[buf], gsems[buf])

    def wait_gather(buf):
        pltpu.make_async_copy(u_hbm.at[gidx_v.at[0]], ins[buf],
                              gsems[buf]).wait()

    def start_scatter(j):
        pltpu.async_copy(out_v, acc_sh.at[sidx_v.at[j]], ssem, add=True)

    def wait_scatter():
        pltpu.make_async_copy(out_v, acc_sh.at[sidx_v.at[0]], ssem).wait()

    def scale(buf, j):
        rin = ins[buf]
        rout = out_v

        @pl.loop(0, CHUNK // L)
        def _(g):
            wv = ew_v.at[j, pl.ds(g * L, L)][...]
            for i in range(L):
                w = lax.broadcast(wv[i], (L,))
                e = g * L + i
                for k in range(D // 32):
                    xb = rin.at[e, pl.ds(k * 32, 32)][...]
                    xi = plsc.bitcast(xb, jnp.int32)
                    lo = plsc.bitcast(xi << 16, jnp.float32)
                    hi = plsc.bitcast(xi & hmask, jnp.float32)
                    rout.at[e, pl.ds(32 * k, L)][...] = lo * w
                    rout.at[e, pl.ds(32 * k + L, L)][...] = hi * w

    @pl.loop(0, CHUNK)
    def _(e):
        for k in range(D // L):
            out_v.at[e, pl.ds(k * L, L)][...] = zero

    @pl.loop(0, ROWS_PER_TILE // CHUNK)
    def _(r):
        pltpu.sync_copy(
            out_v, acc_sh.at[pl.ds(s * ROWS_PER_TILE + r * CHUNK, CHUNK)])

    rem = ROWS_PER_TILE % CHUNK
    if rem:
        pltpu.sync_copy(
            out_v.at[pl.ds(0, rem)],
            acc_sh.at[pl.ds(s * ROWS_PER_TILE + ROWS_PER_TILE - rem, rem)])

    plsc.subcore_barrier()

    @pl.loop(0, NCHUNK // GB)
    def _(b):
        @pl.when(c == 0)
        def _():
            pltpu.sync_copy(dst_hbm.at[s, pl.ds(b * GB, GB)], gidx_v)
            pltpu.sync_copy(src_hbm.at[s, pl.ds(b * GB, GB)], sidx_v)

        @pl.when(c != 0)
        def _():
            pltpu.sync_copy(src_hbm.at[s, pl.ds(b * GB, GB)], gidx_v)
            pltpu.sync_copy(dst_hbm.at[s, pl.ds(b * GB, GB)], sidx_v)

        pltpu.sync_copy(ew_hbm.at[s, pl.ds(b * GB, GB)], ew_v)

        for r in range(NBUF):
            start_gather(r, r)

        @pl.loop(0, GB // NBUF)
        def _(t):
            for r in range(NBUF):
                j = NBUF * t + r
                wait_gather(r)

                if r == 0:
                    @pl.when(t > 0)
                    def _():
                        wait_scatter()
                else:
                    wait_scatter()

                scale(r, j)

                @pl.when(j + NBUF < GB)
                def _():
                    start_gather(r, j + NBUF)

                start_scatter(j)

        wait_scatter()

    plsc.subcore_barrier()

    @pl.when(c == 0)
    def _():
        pltpu.sync_copy(acc_sh.at[pl.ds(s * ROWS_PER_TILE, ROWS_PER_TILE)],
                        seg1_hbm.at[pl.ds(s * ROWS_PER_TILE, ROWS_PER_TILE)])

    @pl.when(c != 0)
    def _():
        pltpu.sync_copy(acc_sh.at[pl.ds(s * ROWS_PER_TILE, ROWS_PER_TILE)],
                        seg2_hbm.at[pl.ds(s * ROWS_PER_TILE, ROWS_PER_TILE)])


_BL = 1280
_BR = 1024


def _degsum_body(part_ref, inv_ref):
    p = part_ref[...]
    deg = jnp.sum(p, axis=0)
    safe = jnp.where(deg > 0.0, deg, 1.0)
    inv_ref[...] = jnp.where(deg > 0.0, lax.rsqrt(safe), 0.0)


_degsum_call = pl.pallas_call(
    _degsum_body,
    grid=(NP // _BL,),
    in_specs=[pl.BlockSpec((NT, 2, _BL), lambda i: (0, 0, i))],
    out_specs=pl.BlockSpec((2, _BL), lambda i: (0, i)),
    out_shape=jax.ShapeDtypeStruct((2, NP), jnp.float32),
)


def _dot(a, b):
    return lax.dot_general(a, b, (((1,), (0,)), ((), ())),
                           precision=lax.Precision.HIGHEST,
                           preferred_element_type=jnp.float32)


def _uv_body(h_ref, w1_ref, w2_ref, cs_ref, u_ref, v_ref):
    h = h_ref[...]
    cs = cs_ref[...]
    u_ref[...] = (_dot(h, w1_ref[...]) * cs[:, 1:2]).astype(jnp.bfloat16)
    v_ref[...] = (_dot(h, w2_ref[...]) * cs[:, 0:1]).astype(jnp.bfloat16)


_uv_call = pl.pallas_call(
    _uv_body,
    grid=(NP // _BR,),
    in_specs=[
        pl.BlockSpec((_BR, D), lambda i: (i, 0)),
        pl.BlockSpec((D, D), lambda i: (0, 0)),
        pl.BlockSpec((D, D), lambda i: (0, 0)),
        pl.BlockSpec((_BR, 2), lambda i: (i, 0)),
    ],
    out_specs=[
        pl.BlockSpec((_BR, D), lambda i: (i, 0)),
        pl.BlockSpec((_BR, D), lambda i: (i, 0)),
    ],
    out_shape=[
        jax.ShapeDtypeStruct((NP, D), jnp.bfloat16),
        jax.ShapeDtypeStruct((NP, D), jnp.bfloat16),
    ],
)


def _layer_h(s1_ref, s2_ref, cs_ref, b1_ref, b2_ref):
    cs = cs_ref[...]
    t1 = cs[:, 0:1] * s1_ref[...] + b1_ref[...]
    t2 = cs[:, 1:2] * s2_ref[...] + b2_ref[...]
    return jnp.maximum(ALPHA * t1 + (1.0 - ALPHA) * t2, 0.0)


def _mid_body(s1_ref, s2_ref, cs_ref, b1_ref, b2_ref, jk_ref, w1_ref, w2_ref,
              jko_ref, u_ref, v_ref):
    h = _layer_h(s1_ref, s2_ref, cs_ref, b1_ref, b2_ref)
    cs = cs_ref[...]
    jko_ref[...] = jnp.maximum(jk_ref[...], h)
    u_ref[...] = (_dot(h, w1_ref[...]) * cs[:, 1:2]).astype(jnp.bfloat16)
    v_ref[...] = (_dot(h, w2_ref[...]) * cs[:, 0:1]).astype(jnp.bfloat16)


_mid_call = pl.pallas_call(
    _mid_body,
    grid=(NP // _BR,),
    in_specs=[
        pl.BlockSpec((_BR, D), lambda i: (i, 0)),
        pl.BlockSpec((_BR, D), lambda i: (i, 0)),
        pl.BlockSpec((_BR, 2), lambda i: (i, 0)),
        pl.BlockSpec((1, D), lambda i: (0, 0)),
        pl.BlockSpec((1, D), lambda i: (0, 0)),
        pl.BlockSpec((_BR, D), lambda i: (i, 0)),
        pl.BlockSpec((D, D), lambda i: (0, 0)),
        pl.BlockSpec((D, D), lambda i: (0, 0)),
    ],
    out_specs=[
        pl.BlockSpec((_BR, D), lambda i: (i, 0)),
        pl.BlockSpec((_BR, D), lambda i: (i, 0)),
        pl.BlockSpec((_BR, D), lambda i: (i, 0)),
    ],
    out_shape=[
        jax.ShapeDtypeStruct((NP, D), jnp.float32),
        jax.ShapeDtypeStruct((NP, D), jnp.bfloat16),
        jax.ShapeDtypeStruct((NP, D), jnp.bfloat16),
    ],
)


def _fin_body(jk_ref, wl_ref, bl_ref, out_ref):
    out_ref[...] = _dot(jk_ref[...], wl_ref[...]) + bl_ref[...]


_fin_call = pl.pallas_call(
    _fin_body,
    grid=(NP // _BR,),
    in_specs=[
        pl.BlockSpec((_BR, D), lambda i: (i, 0)),
        pl.BlockSpec((D, D), lambda i: (0, 0)),
        pl.BlockSpec((1, D), lambda i: (0, 0)),
    ],
    out_specs=pl.BlockSpec((_BR, D), lambda i: (i, 0)),
    out_shape=jax.ShapeDtypeStruct((NP, D), jnp.float32),
)


@jax.jit
def kernel(x, edge_index, edge_weight, W_s2d, b_s2d, W_d2s, b_d2s, W_lin,
           b_lin):
    E = edge_index.shape[1]
    pad = E_PAD - E

    src = jnp.concatenate([edge_index[0], jnp.zeros((pad,), jnp.int32)])
    dst = jnp.concatenate([edge_index[1], jnp.zeros((pad,), jnp.int32)])
    ew = jnp.concatenate([edge_weight, jnp.zeros((pad,), jnp.float32)])
    src3 = src.reshape(NS, NCHUNK, CHUNK)
    dst3 = dst.reshape(NS, NCHUNK, CHUNK)
    ew3 = ew.reshape(NS, NCHUNK, CHUNK)

    xp = jnp.zeros((NP, D), jnp.float32).at[:N].set(x)

    part = _deg_kernel(src3, dst3, ew3)
    inv = _degsum_call(part)
    colscale = inv.T

    b1 = b_s2d.reshape(NUM_LAYERS, 1, D)
    b2 = b_d2s.reshape(NUM_LAYERS, 1, D)

    isig = jnp.asarray(_ISIG)
    W1p = W_s2d[:, :, isig]
    W2p = W_d2s[:, :, isig]

    u, v = _uv_call(xp, W1p[0], W2p[0], colscale)
    jk = jnp.zeros((NP, D), jnp.float32)

    zw = jnp.zeros((1, D, D), jnp.float32)
    W1n = jnp.concatenate([W1p[1:], zw])
    W2n = jnp.concatenate([W2p[1:], zw])

    def body(carry, xs):
        u, v, jk = carry
        w1n, w2n, b1i, b2i = xs
        seg1, seg2 = _edge_kernel(u, v, src3, dst3, ew3)
        jk, u, v = _mid_call(seg1, seg2, colscale, b1i, b2i, jk, w1n, w2n)
        return (u, v, jk), None

    (u, v, jk), _ = lax.scan(body, (u, v, jk), (W1n, W2n, b1, b2))
    out = _fin_call(jk, W_lin, b_lin.reshape(1, D))
    return out[:N]


# speedup vs baseline: 1.1120x; 1.1120x over previous
import dataclasses
import functools

import jax
import jax.numpy as jnp
import numpy as np
from jax import lax
from jax.experimental import pallas as pl
from jax.experimental.pallas import tpu as pltpu
from jax.experimental.pallas import tpu_sc as plsc

N = 10000
D = 128
NUM_LAYERS = 3
ALPHA = 0.5

NC = 2
NS = 16
NT = NC * NS
L = 16

NP = 10240
NACC = 10112
CHUNK = 128
NCHUNK = 160
GB = 8
NBUF = 2
HC = CHUNK // 2
SLAB = NCHUNK * CHUNK
E_PAD = NS * SLAB

ROWS_PER_TILE = NACC // NS

_ISIG = np.empty((D,), np.int32)
for _k in range(D // 32):
    for _j in range(16):
        _ISIG[32 * _k + 2 * _j] = 32 * _k + _j
        _ISIG[32 * _k + 2 * _j + 1] = 32 * _k + 16 + _j

_mesh = plsc.VectorSubcoreMesh(
    core_axis_name="c", subcore_axis_name="s", num_cores=NC, num_subcores=NS
)

_sc_params = pltpu.CompilerParams()
if "needs_layout_passes" in pltpu.CompilerParams.__dataclass_fields__:
    _sc_params = dataclasses.replace(_sc_params, needs_layout_passes=False)
if "use_tc_tiling_on_sc" in pltpu.CompilerParams.__dataclass_fields__:
    _sc_params = dataclasses.replace(_sc_params, use_tc_tiling_on_sc=False)


@functools.partial(
    pl.kernel,
    out_type=jax.ShapeDtypeStruct((NT, 2, NP), jnp.float32),
    mesh=_mesh,
    scratch_types=[
        pltpu.VMEM((NCHUNK, CHUNK), jnp.int32),
        pltpu.VMEM((NCHUNK, CHUNK), jnp.int32),
        pltpu.VMEM((NCHUNK, CHUNK), jnp.float32),
        pltpu.VMEM((NP,), jnp.float32),
        pltpu.VMEM((NP,), jnp.float32),
    ],
    compiler_params=_sc_params,
)
def _deg_kernel(src_hbm, dst_hbm, ew_hbm, part_hbm, src_v, dst_v, ew_v,
                acco_v, acci_v):
    c = lax.axis_index("c")
    s = lax.axis_index("s")
    pltpu.sync_copy(src_hbm.at[s], src_v)
    pltpu.sync_copy(dst_hbm.at[s], dst_v)
    pltpu.sync_copy(ew_hbm.at[s], ew_v)

    zero = jnp.zeros((L,), jnp.float32)

    @pl.loop(0, NP // L)
    def _(i):
        acco_v.at[pl.ds(i * L, L)][...] = zero
        acci_v.at[pl.ds(i * L, L)][...] = zero

    half = NCHUNK // 2

    @pl.loop(0, half)
    def _(jj):
        j = c * half + jj

        @pl.loop(0, CHUNK // L)
        def _(g):
            sv = src_v.at[j, pl.ds(g * L, L)][...]
            dv = dst_v.at[j, pl.ds(g * L, L)][...]
            wv = ew_v.at[j, pl.ds(g * L, L)][...]
            plsc.addupdate_scatter(acco_v, [sv], wv)
            plsc.addupdate_scatter(acci_v, [dv], wv)

    w = c * NS + s
    pltpu.sync_copy(acco_v, part_hbm.at[w, 0])
    pltpu.sync_copy(acci_v, part_hbm.at[w, 1])


@functools.partial(
    pl.kernel,
    out_type=(
        jax.ShapeDtypeStruct((NP, D), jnp.float32),
        jax.ShapeDtypeStruct((NP, D), jnp.float32),
    ),
    mesh=_mesh,
    scratch_types=[
        pltpu.VMEM((GB, 2, HC), jnp.int32),
        pltpu.VMEM((GB, 2, HC), jnp.int32),
        pltpu.VMEM((GB, CHUNK), jnp.float32),
        pltpu.VMEM((CHUNK, D), jnp.bfloat16),
        pltpu.VMEM((CHUNK, D), jnp.bfloat16),
        pltpu.VMEM((CHUNK, D), jnp.float32),
        pltpu.VMEM_SHARED((NACC, D), jnp.float32),
        pltpu.SemaphoreType.DMA,
        pltpu.SemaphoreType.DMA,
        pltpu.SemaphoreType.DMA,
    ],
    compiler_params=_sc_params,
)
def _edge_kernel(u_hbm, v_hbm, src_hbm, dst_hbm, ew_hbm, seg1_hbm, seg2_hbm,
                 gidx_v, sidx_v, ew_v, in_a, in_b, out_v, acc_sh,
                 gsem_a, gsem_b, ssem):
    c = lax.axis_index("c")
    s = lax.axis_index("s")
    ins = (in_a, in_b)
    gsems = (gsem_a, gsem_b)
    zero = jnp.zeros((L,), jnp.float32)
    hmask = jnp.int32(-65536)

    def start_gather(buf, j):
        for h in range(2):
            idx = gidx_v.at[j, h]
            dst = ins[buf].at[pl.ds(h * HC, HC)]

            @pl.when(c == 0)
            def _():
                pltpu.async_copy(u_hbm.at[idx], dst, gsems[buf])

            @pl.when(c != 0)
            def _():
                pltpu.async_copy(v_hbm.at[idx], dst, gsems[buf])

    def wait_gather(buf):
        for h in range(2):
            pltpu.make_async_copy(u_hbm.at[gidx_v.at[0, 0]],
                                  ins[buf].at[pl.ds(h * HC, HC)],
                                  gsems[buf]).wait()

    def start_scatter(j):
        for h in range(2):
            pltpu.async_copy(out_v.at[pl.ds(h * HC, HC)],
                             acc_sh.at[sidx_v.at[j, h]], ssem, add=True)

    def wait_scatter():
        for h in range(2):
            pltpu.make_async_copy(out_v.at[pl.ds(h * HC, HC)],
                                  acc_sh.at[sidx_v.at[0, 0]], ssem).wait()

    def scale(buf, j):
        rin = ins[buf]
        rout = out_v

        @pl.loop(0, CHUNK // L)
        def _(g):
            wv = ew_v.at[j, pl.ds(g * L, L)][...]
            for i in range(L):
                w = lax.broadcast(wv[i], (L,))
                e = g * L + i
                for k in range(D // 32):
                    xb = rin.at[e, pl.ds(k * 32, 32)][...]
                    xi = plsc.bitcast(xb, jnp.int32)
                    lo = plsc.bitcast(xi << 16, jnp.float32)
                    hi = plsc.bitcast(xi & hmask, jnp.float32)
                    rout.at[e, pl.ds(32 * k, L)][...] = lo * w
                    rout.at[e, pl.ds(32 * k + L, L)][...] = hi * w

    @pl.loop(0, CHUNK)
    def _(e):
        for k in range(D // L):
            out_v.at[e, pl.ds(k * L, L)][...] = zero

    @pl.loop(0, ROWS_PER_TILE // CHUNK)
    def _(r):
        pltpu.sync_copy(
            out_v, acc_sh.at[pl.ds(s * ROWS_PER_TILE + r * CHUNK, CHUNK)])

    rem = ROWS_PER_TILE % CHUNK
    if rem:
        pltpu.sync_copy(
            out_v.at[pl.ds(0, rem)],
            acc_sh.at[pl.ds(s * ROWS_PER_TILE + ROWS_PER_TILE - rem, rem)])

    plsc.subcore_barrier()

    @pl.loop(0, NCHUNK // GB)
    def _(b):
        @pl.when(c == 0)
        def _():
            pltpu.sync_copy(dst_hbm.at[s, pl.ds(b * GB, GB)], gidx_v)
            pltpu.sync_copy(src_hbm.at[s, pl.ds(b * GB, GB)], sidx_v)

        @pl.when(c != 0)
        def _():
            pltpu.sync_copy(src_hbm.at[s, pl.ds(b * GB, GB)], gidx_v)
            pltpu.sync_copy(dst_hbm.at[s, pl.ds(b * GB, GB)], sidx_v)

        pltpu.sync_copy(ew_hbm.at[s, pl.ds(b * GB, GB)], ew_v)

        for r in range(NBUF):
            start_gather(r, r)

        @pl.loop(0, GB // NBUF)
        def _(t):
            for r in range(NBUF):
                j = NBUF * t + r
                wait_gather(r)

                if r == 0:
                    @pl.when(t > 0)
                    def _():
                        wait_scatter()
                else:
                    wait_scatter()

                scale(r, j)

                @pl.when(j + NBUF < GB)
                def _():
                    start_gather(r, j + NBUF)

                start_scatter(j)

        wait_scatter()

    plsc.subcore_barrier()

    @pl.when(c == 0)
    def _():
        pltpu.sync_copy(acc_sh.at[pl.ds(s * ROWS_PER_TILE, ROWS_PER_TILE)],
                        seg1_hbm.at[pl.ds(s * ROWS_PER_TILE, ROWS_PER_TILE)])

    @pl.when(c != 0)
    def _():
        pltpu.sync_copy(acc_sh.at[pl.ds(s * ROWS_PER_TILE, ROWS_PER_TILE)],
                        seg2_hbm.at[pl.ds(s * ROWS_PER_TILE, ROWS_PER_TILE)])


_BL = 1280
_BR = 1024


def _degsum_body(part_ref, inv_ref):
    p = part_ref[...]
    deg = jnp.sum(p, axis=0)
    safe = jnp.where(deg > 0.0, deg, 1.0)
    inv_ref[...] = jnp.where(deg > 0.0, lax.rsqrt(safe), 0.0)


_degsum_call = pl.pallas_call(
    _degsum_body,
    grid=(NP // _BL,),
    in_specs=[pl.BlockSpec((NT, 2, _BL), lambda i: (0, 0, i))],
    out_specs=pl.BlockSpec((2, _BL), lambda i: (0, i)),
    out_shape=jax.ShapeDtypeStruct((2, NP), jnp.float32),
)


def _dot(a, b):
    return lax.dot_general(a, b, (((1,), (0,)), ((), ())),
                           precision=lax.Precision.HIGHEST,
                           preferred_element_type=jnp.float32)


def _uv_body(h_ref, w1_ref, w2_ref, cs_ref, u_ref, v_ref):
    h = h_ref[...]
    cs = cs_ref[...]
    u_ref[...] = (_dot(h, w1_ref[...]) * cs[:, 1:2]).astype(jnp.bfloat16)
    v_ref[...] = (_dot(h, w2_ref[...]) * cs[:, 0:1]).astype(jnp.bfloat16)


_uv_call = pl.pallas_call(
    _uv_body,
    grid=(NP // _BR,),
    in_specs=[
        pl.BlockSpec((_BR, D), lambda i: (i, 0)),
        pl.BlockSpec((D, D), lambda i: (0, 0)),
        pl.BlockSpec((D, D), lambda i: (0, 0)),
        pl.BlockSpec((_BR, 2), lambda i: (i, 0)),
    ],
    out_specs=[
        pl.BlockSpec((_BR, D), lambda i: (i, 0)),
        pl.BlockSpec((_BR, D), lambda i: (i, 0)),
    ],
    out_shape=[
        jax.ShapeDtypeStruct((NP, D), jnp.bfloat16),
        jax.ShapeDtypeStruct((NP, D), jnp.bfloat16),
    ],
)


def _layer_h(s1_ref, s2_ref, cs_ref, b1_ref, b2_ref):
    cs = cs_ref[...]
    t1 = cs[:, 0:1] * s1_ref[...] + b1_ref[...]
    t2 = cs[:, 1:2] * s2_ref[...] + b2_ref[...]
    return jnp.maximum(ALPHA * t1 + (1.0 - ALPHA) * t2, 0.0)


def _mid_body(s1_ref, s2_ref, cs_ref, b1_ref, b2_ref, jk_ref, w1_ref, w2_ref,
              jko_ref, u_ref, v_ref):
    h = _layer_h(s1_ref, s2_ref, cs_ref, b1_ref, b2_ref)
    cs = cs_ref[...]
    jko_ref[...] = jnp.maximum(jk_ref[...], h)
    u_ref[...] = (_dot(h, w1_ref[...]) * cs[:, 1:2]).astype(jnp.bfloat16)
    v_ref[...] = (_dot(h, w2_ref[...]) * cs[:, 0:1]).astype(jnp.bfloat16)


_mid_call = pl.pallas_call(
    _mid_body,
    grid=(NP // _BR,),
    in_specs=[
        pl.BlockSpec((_BR, D), lambda i: (i, 0)),
        pl.BlockSpec((_BR, D), lambda i: (i, 0)),
        pl.BlockSpec((_BR, 2), lambda i: (i, 0)),
        pl.BlockSpec((1, D), lambda i: (0, 0)),
        pl.BlockSpec((1, D), lambda i: (0, 0)),
        pl.BlockSpec((_BR, D), lambda i: (i, 0)),
        pl.BlockSpec((D, D), lambda i: (0, 0)),
        pl.BlockSpec((D, D), lambda i: (0, 0)),
    ],
    out_specs=[
        pl.BlockSpec((_BR, D), lambda i: (i, 0)),
        pl.BlockSpec((_BR, D), lambda i: (i, 0)),
        pl.BlockSpec((_BR, D), lambda i: (i, 0)),
    ],
    out_shape=[
        jax.ShapeDtypeStruct((NP, D), jnp.float32),
        jax.ShapeDtypeStruct((NP, D), jnp.bfloat16),
        jax.ShapeDtypeStruct((NP, D), jnp.bfloat16),
    ],
)


def _fin_body(jk_ref, wl_ref, bl_ref, out_ref):
    out_ref[...] = _dot(jk_ref[...], wl_ref[...]) + bl_ref[...]


_fin_call = pl.pallas_call(
    _fin_body,
    grid=(NP // _BR,),
    in_specs=[
        pl.BlockSpec((_BR, D), lambda i: (i, 0)),
        pl.BlockSpec((D, D), lambda i: (0, 0)),
        pl.BlockSpec((1, D), lambda i: (0, 0)),
    ],
    out_specs=pl.BlockSpec((_BR, D), lambda i: (i, 0)),
    out_shape=jax.ShapeDtypeStruct((NP, D), jnp.float32),
)


@jax.jit
def kernel(x, edge_index, edge_weight, W_s2d, b_s2d, W_d2s, b_d2s, W_lin,
           b_lin):
    E = edge_index.shape[1]
    pad = E_PAD - E

    src = jnp.concatenate([edge_index[0], jnp.zeros((pad,), jnp.int32)])
    dst = jnp.concatenate([edge_index[1], jnp.zeros((pad,), jnp.int32)])
    ew = jnp.concatenate([edge_weight, jnp.zeros((pad,), jnp.float32)])
    src3 = src.reshape(NS, NCHUNK, CHUNK)
    dst3 = dst.reshape(NS, NCHUNK, CHUNK)
    ew3 = ew.reshape(NS, NCHUNK, CHUNK)
    src4 = src.reshape(NS, NCHUNK, 2, HC)
    dst4 = dst.reshape(NS, NCHUNK, 2, HC)

    xp = jnp.zeros((NP, D), jnp.float32).at[:N].set(x)

    part = _deg_kernel(src3, dst3, ew3)
    inv = _degsum_call(part)
    colscale = inv.T

    b1 = b_s2d.reshape(NUM_LAYERS, 1, D)
    b2 = b_d2s.reshape(NUM_LAYERS, 1, D)

    isig = jnp.asarray(_ISIG)
    W1p = W_s2d[:, :, isig]
    W2p = W_d2s[:, :, isig]

    u, v = _uv_call(xp, W1p[0], W2p[0], colscale)
    jk = jnp.zeros((NP, D), jnp.float32)

    zw = jnp.zeros((1, D, D), jnp.float32)
    W1n = jnp.concatenate([W1p[1:], zw])
    W2n = jnp.concatenate([W2p[1:], zw])

    def body(carry, xs):
        u, v, jk = carry
        w1n, w2n, b1i, b2i = xs
        seg1, seg2 = _edge_kernel(u, v, src4, dst4, ew3)
        jk, u, v = _mid_call(seg1, seg2, colscale, b1i, b2i, jk, w1n, w2n)
        return (u, v, jk), None

    (u, v, jk), _ = lax.scan(body, (u, v, jk), (W1n, W2n, b1, b2))
    out = _fin_call(jk, W_lin, b_lin.reshape(1, D))
    return out[:N]


# speedup vs baseline: 1.1863x; 1.0669x over previous
import dataclasses
import functools

import jax
import jax.numpy as jnp
import numpy as np
from jax import lax
from jax.experimental import pallas as pl
from jax.experimental.pallas import tpu as pltpu
from jax.experimental.pallas import tpu_sc as plsc

N = 10000
D = 128
NUM_LAYERS = 3
ALPHA = 0.5

NC = 2
NS = 16
NT = NC * NS
L = 16

NP = 10240
NACC = 10112
CHUNK = 112
NCHUNK = 184
GB = 8
NBUF = 2
HC = CHUNK // 2
SLAB = NCHUNK * CHUNK
E_PAD = NS * SLAB

ROWS_PER_TILE = NACC // NS

_ISIG = np.empty((D,), np.int32)
for _k in range(D // 32):
    for _j in range(16):
        _ISIG[32 * _k + 2 * _j] = 32 * _k + _j
        _ISIG[32 * _k + 2 * _j + 1] = 32 * _k + 16 + _j

_mesh = plsc.VectorSubcoreMesh(
    core_axis_name="c", subcore_axis_name="s", num_cores=NC, num_subcores=NS
)

_sc_params = pltpu.CompilerParams()
if "needs_layout_passes" in pltpu.CompilerParams.__dataclass_fields__:
    _sc_params = dataclasses.replace(_sc_params, needs_layout_passes=False)
if "use_tc_tiling_on_sc" in pltpu.CompilerParams.__dataclass_fields__:
    _sc_params = dataclasses.replace(_sc_params, use_tc_tiling_on_sc=False)


@functools.partial(
    pl.kernel,
    out_type=jax.ShapeDtypeStruct((NT, 2, NP), jnp.float32),
    mesh=_mesh,
    scratch_types=[
        pltpu.VMEM((NCHUNK, CHUNK), jnp.int32),
        pltpu.VMEM((NCHUNK, CHUNK), jnp.int32),
        pltpu.VMEM((NCHUNK, CHUNK), jnp.float32),
        pltpu.VMEM((NP,), jnp.float32),
        pltpu.VMEM((NP,), jnp.float32),
    ],
    compiler_params=_sc_params,
)
def _deg_kernel(src_hbm, dst_hbm, ew_hbm, part_hbm, src_v, dst_v, ew_v,
                acco_v, acci_v):
    c = lax.axis_index("c")
    s = lax.axis_index("s")
    pltpu.sync_copy(src_hbm.at[s], src_v)
    pltpu.sync_copy(dst_hbm.at[s], dst_v)
    pltpu.sync_copy(ew_hbm.at[s], ew_v)

    zero = jnp.zeros((L,), jnp.float32)

    @pl.loop(0, NP // L)
    def _(i):
        acco_v.at[pl.ds(i * L, L)][...] = zero
        acci_v.at[pl.ds(i * L, L)][...] = zero

    half = NCHUNK // 2

    @pl.loop(0, half)
    def _(jj):
        j = c * half + jj

        @pl.loop(0, CHUNK // L)
        def _(g):
            sv = src_v.at[j, pl.ds(g * L, L)][...]
            dv = dst_v.at[j, pl.ds(g * L, L)][...]
            wv = ew_v.at[j, pl.ds(g * L, L)][...]
            plsc.addupdate_scatter(acco_v, [sv], wv)
            plsc.addupdate_scatter(acci_v, [dv], wv)

    w = c * NS + s
    pltpu.sync_copy(acco_v, part_hbm.at[w, 0])
    pltpu.sync_copy(acci_v, part_hbm.at[w, 1])


@functools.partial(
    pl.kernel,
    out_type=(
        jax.ShapeDtypeStruct((NP, D), jnp.float32),
        jax.ShapeDtypeStruct((NP, D), jnp.float32),
    ),
    mesh=_mesh,
    scratch_types=[
        pltpu.VMEM((GB, 2, HC), jnp.int32),
        pltpu.VMEM((GB, 2, HC), jnp.int32),
        pltpu.VMEM((GB, CHUNK), jnp.float32),
        pltpu.VMEM((CHUNK, D), jnp.bfloat16),
        pltpu.VMEM((CHUNK, D), jnp.bfloat16),
        pltpu.VMEM((CHUNK, D), jnp.float32),
        pltpu.VMEM((CHUNK, D), jnp.float32),
        pltpu.VMEM_SHARED((NACC, D), jnp.float32),
        pltpu.SemaphoreType.DMA,
        pltpu.SemaphoreType.DMA,
        pltpu.SemaphoreType.DMA,
        pltpu.SemaphoreType.DMA,
    ],
    compiler_params=_sc_params,
)
def _edge_kernel(u_hbm, v_hbm, src_hbm, dst_hbm, ew_hbm, seg1_hbm, seg2_hbm,
                 gidx_v, sidx_v, ew_v, in_a, in_b, out_a, out_b, acc_sh,
                 gsem_a, gsem_b, ssem_a, ssem_b):
    c = lax.axis_index("c")
    s = lax.axis_index("s")
    ins = (in_a, in_b)
    outs = (out_a, out_b)
    gsems = (gsem_a, gsem_b)
    ssems = (ssem_a, ssem_b)
    zero = jnp.zeros((L,), jnp.float32)
    hmask = jnp.int32(-65536)

    def start_gather(buf, j):
        for h in range(2):
            idx = gidx_v.at[j, h]
            dst = ins[buf].at[pl.ds(h * HC, HC)]

            @pl.when(c == 0)
            def _():
                pltpu.async_copy(u_hbm.at[idx], dst, gsems[buf])

            @pl.when(c != 0)
            def _():
                pltpu.async_copy(v_hbm.at[idx], dst, gsems[buf])

    def wait_gather(buf):
        for h in range(2):
            pltpu.make_async_copy(u_hbm.at[gidx_v.at[0, 0]],
                                  ins[buf].at[pl.ds(h * HC, HC)],
                                  gsems[buf]).wait()

    def start_scatter(buf, j):
        for h in range(2):
            pltpu.async_copy(outs[buf].at[pl.ds(h * HC, HC)],
                             acc_sh.at[sidx_v.at[j, h]], ssems[buf], add=True)

    def wait_scatter(buf):
        for h in range(2):
            pltpu.make_async_copy(outs[buf].at[pl.ds(h * HC, HC)],
                                  acc_sh.at[sidx_v.at[0, 0]], ssems[buf]).wait()

    def scale(buf, j):
        rin = ins[buf]
        rout = outs[buf]

        @pl.loop(0, CHUNK // L)
        def _(g):
            wv = ew_v.at[j, pl.ds(g * L, L)][...]
            for i in range(L):
                w = lax.broadcast(wv[i], (L,))
                e = g * L + i
                for k in range(D // 32):
                    xb = rin.at[e, pl.ds(k * 32, 32)][...]
                    xi = plsc.bitcast(xb, jnp.int32)
                    lo = plsc.bitcast(xi << 16, jnp.float32)
                    hi = plsc.bitcast(xi & hmask, jnp.float32)
                    rout.at[e, pl.ds(32 * k, L)][...] = lo * w
                    rout.at[e, pl.ds(32 * k + L, L)][...] = hi * w

    @pl.loop(0, CHUNK)
    def _(e):
        for k in range(D // L):
            out_a.at[e, pl.ds(k * L, L)][...] = zero

    @pl.loop(0, ROWS_PER_TILE // CHUNK)
    def _(r):
        pltpu.sync_copy(
            out_a, acc_sh.at[pl.ds(s * ROWS_PER_TILE + r * CHUNK, CHUNK)])

    rem = ROWS_PER_TILE % CHUNK
    if rem:
        pltpu.sync_copy(
            out_a.at[pl.ds(0, rem)],
            acc_sh.at[pl.ds(s * ROWS_PER_TILE + ROWS_PER_TILE - rem, rem)])

    plsc.subcore_barrier()

    @pl.loop(0, NCHUNK // GB)
    def _(b):
        @pl.when(c == 0)
        def _():
            pltpu.sync_copy(dst_hbm.at[s, pl.ds(b * GB, GB)], gidx_v)
            pltpu.sync_copy(src_hbm.at[s, pl.ds(b * GB, GB)], sidx_v)

        @pl.when(c != 0)
        def _():
            pltpu.sync_copy(src_hbm.at[s, pl.ds(b * GB, GB)], gidx_v)
            pltpu.sync_copy(dst_hbm.at[s, pl.ds(b * GB, GB)], sidx_v)

        pltpu.sync_copy(ew_hbm.at[s, pl.ds(b * GB, GB)], ew_v)

        for r in range(NBUF):
            start_gather(r, r)

        @pl.loop(0, GB // NBUF)
        def _(t):
            for r in range(NBUF):
                j = NBUF * t + r
                wait_gather(r)

                @pl.when(t > 0)
                def _():
                    wait_scatter(r)

                scale(r, j)

                @pl.when(j + NBUF < GB)
                def _():
                    start_gather(r, j + NBUF)

                start_scatter(r, j)

        wait_scatter(0)
        wait_scatter(1)

    plsc.subcore_barrier()

    @pl.when(c == 0)
    def _():
        pltpu.sync_copy(acc_sh.at[pl.ds(s * ROWS_PER_TILE, ROWS_PER_TILE)],
                        seg1_hbm.at[pl.ds(s * ROWS_PER_TILE, ROWS_PER_TILE)])

    @pl.when(c != 0)
    def _():
        pltpu.sync_copy(acc_sh.at[pl.ds(s * ROWS_PER_TILE, ROWS_PER_TILE)],
                        seg2_hbm.at[pl.ds(s * ROWS_PER_TILE, ROWS_PER_TILE)])


_BL = 1280
_BR = 1024


def _degsum_body(part_ref, inv_ref):
    p = part_ref[...]
    deg = jnp.sum(p, axis=0)
    safe = jnp.where(deg > 0.0, deg, 1.0)
    inv_ref[...] = jnp.where(deg > 0.0, lax.rsqrt(safe), 0.0)


_degsum_call = pl.pallas_call(
    _degsum_body,
    grid=(NP // _BL,),
    in_specs=[pl.BlockSpec((NT, 2, _BL), lambda i: (0, 0, i))],
    out_specs=pl.BlockSpec((2, _BL), lambda i: (0, i)),
    out_shape=jax.ShapeDtypeStruct((2, NP), jnp.float32),
)


def _dot(a, b):
    return lax.dot_general(a, b, (((1,), (0,)), ((), ())),
                           precision=lax.Precision.HIGHEST,
                           preferred_element_type=jnp.float32)


def _uv_body(h_ref, w1_ref, w2_ref, cs_ref, u_ref, v_ref):
    h = h_ref[...]
    cs = cs_ref[...]
    u_ref[...] = (_dot(h, w1_ref[...]) * cs[:, 1:2]).astype(jnp.bfloat16)
    v_ref[...] = (_dot(h, w2_ref[...]) * cs[:, 0:1]).astype(jnp.bfloat16)


_uv_call = pl.pallas_call(
    _uv_body,
    grid=(NP // _BR,),
    in_specs=[
        pl.BlockSpec((_BR, D), lambda i: (i, 0)),
        pl.BlockSpec((D, D), lambda i: (0, 0)),
        pl.BlockSpec((D, D), lambda i: (0, 0)),
        pl.BlockSpec((_BR, 2), lambda i: (i, 0)),
    ],
    out_specs=[
        pl.BlockSpec((_BR, D), lambda i: (i, 0)),
        pl.BlockSpec((_BR, D), lambda i: (i, 0)),
    ],
    out_shape=[
        jax.ShapeDtypeStruct((NP, D), jnp.bfloat16),
        jax.ShapeDtypeStruct((NP, D), jnp.bfloat16),
    ],
)


def _layer_h(s1_ref, s2_ref, cs_ref, b1_ref, b2_ref):
    cs = cs_ref[...]
    t1 = cs[:, 0:1] * s1_ref[...] + b1_ref[...]
    t2 = cs[:, 1:2] * s2_ref[...] + b2_ref[...]
    return jnp.maximum(ALPHA * t1 + (1.0 - ALPHA) * t2, 0.0)


def _mid_body(s1_ref, s2_ref, cs_ref, b1_ref, b2_ref, jk_ref, w1_ref, w2_ref,
              jko_ref, u_ref, v_ref):
    h = _layer_h(s1_ref, s2_ref, cs_ref, b1_ref, b2_ref)
    cs = cs_ref[...]
    jko_ref[...] = jnp.maximum(jk_ref[...], h)
    u_ref[...] = (_dot(h, w1_ref[...]) * cs[:, 1:2]).astype(jnp.bfloat16)
    v_ref[...] = (_dot(h, w2_ref[...]) * cs[:, 0:1]).astype(jnp.bfloat16)


_mid_call = pl.pallas_call(
    _mid_body,
    grid=(NP // _BR,),
    in_specs=[
        pl.BlockSpec((_BR, D), lambda i: (i, 0)),
        pl.BlockSpec((_BR, D), lambda i: (i, 0)),
        pl.BlockSpec((_BR, 2), lambda i: (i, 0)),
        pl.BlockSpec((1, D), lambda i: (0, 0)),
        pl.BlockSpec((1, D), lambda i: (0, 0)),
        pl.BlockSpec((_BR, D), lambda i: (i, 0)),
        pl.BlockSpec((D, D), lambda i: (0, 0)),
        pl.BlockSpec((D, D), lambda i: (0, 0)),
    ],
    out_specs=[
        pl.BlockSpec((_BR, D), lambda i: (i, 0)),
        pl.BlockSpec((_BR, D), lambda i: (i, 0)),
        pl.BlockSpec((_BR, D), lambda i: (i, 0)),
    ],
    out_shape=[
        jax.ShapeDtypeStruct((NP, D), jnp.float32),
        jax.ShapeDtypeStruct((NP, D), jnp.bfloat16),
        jax.ShapeDtypeStruct((NP, D), jnp.bfloat16),
    ],
)


def _fin_body(jk_ref, wl_ref, bl_ref, out_ref):
    out_ref[...] = _dot(jk_ref[...], wl_ref[...]) + bl_ref[...]


_fin_call = pl.pallas_call(
    _fin_body,
    grid=(NP // _BR,),
    in_specs=[
        pl.BlockSpec((_BR, D), lambda i: (i, 0)),
        pl.BlockSpec((D, D), lambda i: (0, 0)),
        pl.BlockSpec((1, D), lambda i: (0, 0)),
    ],
    out_specs=pl.BlockSpec((_BR, D), lambda i: (i, 0)),
    out_shape=jax.ShapeDtypeStruct((NP, D), jnp.float32),
)


@jax.jit
def kernel(x, edge_index, edge_weight, W_s2d, b_s2d, W_d2s, b_d2s, W_lin,
           b_lin):
    E = edge_index.shape[1]
    pad = E_PAD - E

    src = jnp.concatenate([edge_index[0], jnp.zeros((pad,), jnp.int32)])
    dst = jnp.concatenate([edge_index[1], jnp.zeros((pad,), jnp.int32)])
    ew = jnp.concatenate([edge_weight, jnp.zeros((pad,), jnp.float32)])
    src3 = src.reshape(NS, NCHUNK, CHUNK)
    dst3 = dst.reshape(NS, NCHUNK, CHUNK)
    ew3 = ew.reshape(NS, NCHUNK, CHUNK)
    src4 = src.reshape(NS, NCHUNK, 2, HC)
    dst4 = dst.reshape(NS, NCHUNK, 2, HC)

    xp = jnp.zeros((NP, D), jnp.float32).at[:N].set(x)

    part = _deg_kernel(src3, dst3, ew3)
    inv = _degsum_call(part)
    colscale = inv.T

    b1 = b_s2d.reshape(NUM_LAYERS, 1, D)
    b2 = b_d2s.reshape(NUM_LAYERS, 1, D)

    isig = jnp.asarray(_ISIG)
    W1p = W_s2d[:, :, isig]
    W2p = W_d2s[:, :, isig]

    u, v = _uv_call(xp, W1p[0], W2p[0], colscale)
    jk = jnp.zeros((NP, D), jnp.float32)

    zw = jnp.zeros((1, D, D), jnp.float32)
    W1n = jnp.concatenate([W1p[1:], zw])
    W2n = jnp.concatenate([W2p[1:], zw])

    def body(carry, xs):
        u, v, jk = carry
        w1n, w2n, b1i, b2i = xs
        seg1, seg2 = _edge_kernel(u, v, src4, dst4, ew3)
        jk, u, v = _mid_call(seg1, seg2, colscale, b1i, b2i, jk, w1n, w2n)
        return (u, v, jk), None

    (u, v, jk), _ = lax.scan(body, (u, v, jk), (W1n, W2n, b1, b2))
    out = _fin_call(jk, W_lin, b_lin.reshape(1, D))
    return out[:N]


# speedup vs baseline: 1.3206x; 1.1132x over previous
import dataclasses
import functools

import jax
import jax.numpy as jnp
import numpy as np
from jax import lax
from jax.experimental import pallas as pl
from jax.experimental.pallas import tpu as pltpu
from jax.experimental.pallas import tpu_sc as plsc

N = 10000
D = 128
NUM_LAYERS = 3
ALPHA = 0.5

NC = 2
NS = 16
NT = NC * NS
L = 16

NP = 10240
NACC = 10112
CHUNK = 112
NCHUNK = 184
GB = 8
NBUF = 2
HC = CHUNK // 2
SLAB = NCHUNK * CHUNK
E_PAD = NS * SLAB

ROWS_PER_TILE = NACC // NS

_ISIG = np.empty((D,), np.int32)
for _k in range(D // 32):
    for _j in range(16):
        _ISIG[32 * _k + 2 * _j] = 32 * _k + _j
        _ISIG[32 * _k + 2 * _j + 1] = 32 * _k + 16 + _j

_mesh = plsc.VectorSubcoreMesh(
    core_axis_name="c", subcore_axis_name="s", num_cores=NC, num_subcores=NS
)

_sc_params = pltpu.CompilerParams()
if "needs_layout_passes" in pltpu.CompilerParams.__dataclass_fields__:
    _sc_params = dataclasses.replace(_sc_params, needs_layout_passes=False)
if "use_tc_tiling_on_sc" in pltpu.CompilerParams.__dataclass_fields__:
    _sc_params = dataclasses.replace(_sc_params, use_tc_tiling_on_sc=False)


@functools.partial(
    pl.kernel,
    out_type=jax.ShapeDtypeStruct((NT, 2, NP), jnp.float32),
    mesh=_mesh,
    scratch_types=[
        pltpu.VMEM((NCHUNK, CHUNK), jnp.int32),
        pltpu.VMEM((NCHUNK, CHUNK), jnp.int32),
        pltpu.VMEM((NCHUNK, CHUNK), jnp.float32),
        pltpu.VMEM((NP,), jnp.float32),
        pltpu.VMEM((NP,), jnp.float32),
    ],
    compiler_params=_sc_params,
)
def _deg_kernel(src_hbm, dst_hbm, ew_hbm, part_hbm, src_v, dst_v, ew_v,
                acco_v, acci_v):
    c = lax.axis_index("c")
    s = lax.axis_index("s")
    pltpu.sync_copy(src_hbm.at[s], src_v)
    pltpu.sync_copy(dst_hbm.at[s], dst_v)
    pltpu.sync_copy(ew_hbm.at[s], ew_v)

    zero = jnp.zeros((L,), jnp.float32)

    @pl.loop(0, NP // L)
    def _(i):
        acco_v.at[pl.ds(i * L, L)][...] = zero
        acci_v.at[pl.ds(i * L, L)][...] = zero

    half = NCHUNK // 2

    @pl.loop(0, half)
    def _(jj):
        j = c * half + jj

        @pl.loop(0, CHUNK // L)
        def _(g):
            sv = src_v.at[j, pl.ds(g * L, L)][...]
            dv = dst_v.at[j, pl.ds(g * L, L)][...]
            wv = ew_v.at[j, pl.ds(g * L, L)][...]
            plsc.addupdate_scatter(acco_v, [sv], wv)
            plsc.addupdate_scatter(acci_v, [dv], wv)

    w = c * NS + s
    pltpu.sync_copy(acco_v, part_hbm.at[w, 0])
    pltpu.sync_copy(acci_v, part_hbm.at[w, 1])


@functools.partial(
    pl.kernel,
    out_type=(
        jax.ShapeDtypeStruct((NP, D), jnp.float32),
        jax.ShapeDtypeStruct((NP, D), jnp.float32),
    ),
    mesh=_mesh,
    scratch_types=[
        pltpu.VMEM((2, GB, 2, HC), jnp.int32),
        pltpu.VMEM((2, GB, 2, HC), jnp.int32),
        pltpu.VMEM((2, GB, CHUNK), jnp.float32),
        pltpu.VMEM((CHUNK, D), jnp.bfloat16),
        pltpu.VMEM((CHUNK, D), jnp.bfloat16),
        pltpu.VMEM((CHUNK, D), jnp.float32),
        pltpu.VMEM((CHUNK, D), jnp.float32),
        pltpu.VMEM_SHARED((NACC, D), jnp.float32),
        pltpu.SemaphoreType.DMA,
        pltpu.SemaphoreType.DMA,
        pltpu.SemaphoreType.DMA,
        pltpu.SemaphoreType.DMA,
        pltpu.SemaphoreType.DMA,
        pltpu.SemaphoreType.DMA,
    ],
    compiler_params=_sc_params,
)
def _edge_kernel(u_hbm, v_hbm, src_hbm, dst_hbm, ew_hbm, seg1_hbm, seg2_hbm,
                 gidx_v, sidx_v, ew_v, in_a, in_b, out_a, out_b, acc_sh,
                 gsem_a, gsem_b, ssem_a, ssem_b, isem_a, isem_b):
    c = lax.axis_index("c")
    s = lax.axis_index("s")
    ins = (in_a, in_b)
    outs = (out_a, out_b)
    gsems = (gsem_a, gsem_b)
    ssems = (ssem_a, ssem_b)
    zero = jnp.zeros((L,), jnp.float32)
    hmask = jnp.int32(-65536)

    def start_gather(par, buf, j):
        for h in range(2):
            idx = gidx_v.at[par, j, h]
            dst = ins[buf].at[pl.ds(h * HC, HC)]

            @pl.when(c == 0)
            def _():
                pltpu.async_copy(u_hbm.at[idx], dst, gsems[buf])

            @pl.when(c != 0)
            def _():
                pltpu.async_copy(v_hbm.at[idx], dst, gsems[buf])

    def wait_gather(buf):
        for h in range(2):
            pltpu.make_async_copy(u_hbm.at[gidx_v.at[0, 0, 0]],
                                  ins[buf].at[pl.ds(h * HC, HC)],
                                  gsems[buf]).wait()

    def start_scatter(par, buf, j):
        for h in range(2):
            pltpu.async_copy(outs[buf].at[pl.ds(h * HC, HC)],
                             acc_sh.at[sidx_v.at[par, j, h]], ssems[buf],
                             add=True)

    def wait_scatter(buf):
        for h in range(2):
            pltpu.make_async_copy(outs[buf].at[pl.ds(h * HC, HC)],
                                  acc_sh.at[sidx_v.at[0, 0, 0]],
                                  ssems[buf]).wait()

    def start_stage(par, b):
        isem = (isem_a, isem_b)[par]
        blk = pl.ds(b * GB, GB)

        @pl.when(c == 0)
        def _():
            pltpu.async_copy(dst_hbm.at[s, blk], gidx_v.at[par], isem)
            pltpu.async_copy(src_hbm.at[s, blk], sidx_v.at[par], isem)

        @pl.when(c != 0)
        def _():
            pltpu.async_copy(src_hbm.at[s, blk], gidx_v.at[par], isem)
            pltpu.async_copy(dst_hbm.at[s, blk], sidx_v.at[par], isem)

        pltpu.async_copy(ew_hbm.at[s, blk], ew_v.at[par], isem)

    def wait_stage(par):
        isem = (isem_a, isem_b)[par]
        pltpu.make_async_copy(dst_hbm.at[s, pl.ds(0, GB)], gidx_v.at[par],
                              isem).wait()
        pltpu.make_async_copy(src_hbm.at[s, pl.ds(0, GB)], sidx_v.at[par],
                              isem).wait()
        pltpu.make_async_copy(ew_hbm.at[s, pl.ds(0, GB)], ew_v.at[par],
                              isem).wait()

    def scale(par, buf, j):
        rin = ins[buf]
        rout = outs[buf]

        @pl.loop(0, CHUNK // L)
        def _(g):
            wv = ew_v.at[par, j, pl.ds(g * L, L)][...]
            for i in range(L):
                w = lax.broadcast(wv[i], (L,))
                e = g * L + i
                for k in range(D // 32):
                    xb = rin.at[e, pl.ds(k * 32, 32)][...]
                    xi = plsc.bitcast(xb, jnp.int32)
                    lo = plsc.bitcast(xi << 16, jnp.float32)
                    hi = plsc.bitcast(xi & hmask, jnp.float32)
                    rout.at[e, pl.ds(32 * k, L)][...] = lo * w
                    rout.at[e, pl.ds(32 * k + L, L)][...] = hi * w

    @pl.loop(0, CHUNK)
    def _(e):
        for k in range(D // L):
            out_a.at[e, pl.ds(k * L, L)][...] = zero

    @pl.loop(0, ROWS_PER_TILE // CHUNK)
    def _(r):
        pltpu.sync_copy(
            out_a, acc_sh.at[pl.ds(s * ROWS_PER_TILE + r * CHUNK, CHUNK)])

    rem = ROWS_PER_TILE % CHUNK
    if rem:
        pltpu.sync_copy(
            out_a.at[pl.ds(0, rem)],
            acc_sh.at[pl.ds(s * ROWS_PER_TILE + ROWS_PER_TILE - rem, rem)])

    plsc.subcore_barrier()

    NB = NCHUNK // GB
    start_stage(0, 0)

    @pl.loop(0, NB)
    def _(b):
        par = b % 2
        first = b == 0
        wait_stage_par = par

        @pl.when(par == 0)
        def _():
            wait_stage(0)

        @pl.when(par == 1)
        def _():
            wait_stage(1)

        @pl.when(b + 1 < NB)
        def _():
            @pl.when(par == 0)
            def _():
                start_stage(1, b + 1)

            @pl.when(par == 1)
            def _():
                start_stage(0, b + 1)

        for r in range(NBUF):
            @pl.when(jnp.logical_not(first))
            def _():
                wait_scatter(r)

            start_gather(par, r, r)

        @pl.loop(0, GB // NBUF)
        def _(t):
            for r in range(NBUF):
                j = NBUF * t + r
                wait_gather(r)

                @pl.when(t > 0)
                def _():
                    wait_scatter(r)

                scale(par, r, j)

                @pl.when(j + NBUF < GB)
                def _():
                    start_gather(par, r, j + NBUF)

                start_scatter(par, r, j)

    wait_scatter(0)
    wait_scatter(1)

    plsc.subcore_barrier()

    @pl.when(c == 0)
    def _():
        pltpu.sync_copy(acc_sh.at[pl.ds(s * ROWS_PER_TILE, ROWS_PER_TILE)],
                        seg1_hbm.at[pl.ds(s * ROWS_PER_TILE, ROWS_PER_TILE)])

    @pl.when(c != 0)
    def _():
        pltpu.sync_copy(acc_sh.at[pl.ds(s * ROWS_PER_TILE, ROWS_PER_TILE)],
                        seg2_hbm.at[pl.ds(s * ROWS_PER_TILE, ROWS_PER_TILE)])


_BL = 1280
_BR = 1024


def _degsum_body(part_ref, inv_ref):
    p = part_ref[...]
    deg = jnp.sum(p, axis=0)
    safe = jnp.where(deg > 0.0, deg, 1.0)
    inv_ref[...] = jnp.where(deg > 0.0, lax.rsqrt(safe), 0.0)


_degsum_call = pl.pallas_call(
    _degsum_body,
    grid=(NP // _BL,),
    in_specs=[pl.BlockSpec((NT, 2, _BL), lambda i: (0, 0, i))],
    out_specs=pl.BlockSpec((2, _BL), lambda i: (0, i)),
    out_shape=jax.ShapeDtypeStruct((2, NP), jnp.float32),
)


def _dot(a, b):
    return lax.dot_general(a, b, (((1,), (0,)), ((), ())),
                           precision=lax.Precision.HIGHEST,
                           preferred_element_type=jnp.float32)


def _uv_body(h_ref, w1_ref, w2_ref, cs_ref, u_ref, v_ref):
    h = h_ref[...]
    cs = cs_ref[...]
    u_ref[...] = (_dot(h, w1_ref[...]) * cs[:, 1:2]).astype(jnp.bfloat16)
    v_ref[...] = (_dot(h, w2_ref[...]) * cs[:, 0:1]).astype(jnp.bfloat16)


_uv_call = pl.pallas_call(
    _uv_body,
    grid=(NP // _BR,),
    in_specs=[
        pl.BlockSpec((_BR, D), lambda i: (i, 0)),
        pl.BlockSpec((D, D), lambda i: (0, 0)),
        pl.BlockSpec((D, D), lambda i: (0, 0)),
        pl.BlockSpec((_BR, 2), lambda i: (i, 0)),
    ],
    out_specs=[
        pl.BlockSpec((_BR, D), lambda i: (i, 0)),
        pl.BlockSpec((_BR, D), lambda i: (i, 0)),
    ],
    out_shape=[
        jax.ShapeDtypeStruct((NP, D), jnp.bfloat16),
        jax.ShapeDtypeStruct((NP, D), jnp.bfloat16),
    ],
)


def _layer_h(s1_ref, s2_ref, cs_ref, b1_ref, b2_ref):
    cs = cs_ref[...]
    t1 = cs[:, 0:1] * s1_ref[...] + b1_ref[...]
    t2 = cs[:, 1:2] * s2_ref[...] + b2_ref[...]
    return jnp.maximum(ALPHA * t1 + (1.0 - ALPHA) * t2, 0.0)


def _mid_body(s1_ref, s2_ref, cs_ref, b1_ref, b2_ref, jk_ref, w1_ref, w2_ref,
              jko_ref, u_ref, v_ref):
    h = _layer_h(s1_ref, s2_ref, cs_ref, b1_ref, b2_ref)
    cs = cs_ref[...]
    jko_ref[...] = jnp.maximum(jk_ref[...], h)
    u_ref[...] = (_dot(h, w1_ref[...]) * cs[:, 1:2]).astype(jnp.bfloat16)
    v_ref[...] = (_dot(h, w2_ref[...]) * cs[:, 0:1]).astype(jnp.bfloat16)


_mid_call = pl.pallas_call(
    _mid_body,
    grid=(NP // _BR,),
    in_specs=[
        pl.BlockSpec((_BR, D), lambda i: (i, 0)),
        pl.BlockSpec((_BR, D), lambda i: (i, 0)),
        pl.BlockSpec((_BR, 2), lambda i: (i, 0)),
        pl.BlockSpec((1, D), lambda i: (0, 0)),
        pl.BlockSpec((1, D), lambda i: (0, 0)),
        pl.BlockSpec((_BR, D), lambda i: (i, 0)),
        pl.BlockSpec((D, D), lambda i: (0, 0)),
        pl.BlockSpec((D, D), lambda i: (0, 0)),
    ],
    out_specs=[
        pl.BlockSpec((_BR, D), lambda i: (i, 0)),
        pl.BlockSpec((_BR, D), lambda i: (i, 0)),
        pl.BlockSpec((_BR, D), lambda i: (i, 0)),
    ],
    out_shape=[
        jax.ShapeDtypeStruct((NP, D), jnp.float32),
        jax.ShapeDtypeStruct((NP, D), jnp.bfloat16),
        jax.ShapeDtypeStruct((NP, D), jnp.bfloat16),
    ],
)


def _fin_body(jk_ref, wl_ref, bl_ref, out_ref):
    out_ref[...] = _dot(jk_ref[...], wl_ref[...]) + bl_ref[...]


_fin_call = pl.pallas_call(
    _fin_body,
    grid=(NP // _BR,),
    in_specs=[
        pl.BlockSpec((_BR, D), lambda i: (i, 0)),
        pl.BlockSpec((D, D), lambda i: (0, 0)),
        pl.BlockSpec((1, D), lambda i: (0, 0)),
    ],
    out_specs=pl.BlockSpec((_BR, D), lambda i: (i, 0)),
    out_shape=jax.ShapeDtypeStruct((NP, D), jnp.float32),
)


@jax.jit
def kernel(x, edge_index, edge_weight, W_s2d, b_s2d, W_d2s, b_d2s, W_lin,
           b_lin):
    E = edge_index.shape[1]
    pad = E_PAD - E

    src = jnp.concatenate([edge_index[0], jnp.zeros((pad,), jnp.int32)])
    dst = jnp.concatenate([edge_index[1], jnp.zeros((pad,), jnp.int32)])
    ew = jnp.concatenate([edge_weight, jnp.zeros((pad,), jnp.float32)])
    src3 = src.reshape(NS, NCHUNK, CHUNK)
    dst3 = dst.reshape(NS, NCHUNK, CHUNK)
    ew3 = ew.reshape(NS, NCHUNK, CHUNK)
    src4 = src.reshape(NS, NCHUNK, 2, HC)
    dst4 = dst.reshape(NS, NCHUNK, 2, HC)

    xp = jnp.zeros((NP, D), jnp.float32).at[:N].set(x)

    part = _deg_kernel(src3, dst3, ew3)
    inv = _degsum_call(part)
    colscale = inv.T

    b1 = b_s2d.reshape(NUM_LAYERS, 1, D)
    b2 = b_d2s.reshape(NUM_LAYERS, 1, D)

    isig = jnp.asarray(_ISIG)
    W1p = W_s2d[:, :, isig]
    W2p = W_d2s[:, :, isig]

    u, v = _uv_call(xp, W1p[0], W2p[0], colscale)
    jk = jnp.zeros((NP, D), jnp.float32)

    zw = jnp.zeros((1, D, D), jnp.float32)
    W1n = jnp.concatenate([W1p[1:], zw])
    W2n = jnp.concatenate([W2p[1:], zw])

    def body(carry, xs):
        u, v, jk = carry
        w1n, w2n, b1i, b2i = xs
        seg1, seg2 = _edge_kernel(u, v, src4, dst4, ew3)
        jk, u, v = _mid_call(seg1, seg2, colscale, b1i, b2i, jk, w1n, w2n)
        return (u, v, jk), None

    (u, v, jk), _ = lax.scan(body, (u, v, jk), (W1n, W2n, b1, b2))
    out = _fin_call(jk, W_lin, b_lin.reshape(1, D))
    return out[:N]


# speedup vs baseline: 1.3211x; 1.0004x over previous
import dataclasses
import functools

import jax
import jax.numpy as jnp
import numpy as np
from jax import lax
from jax.experimental import pallas as pl
from jax.experimental.pallas import tpu as pltpu
from jax.experimental.pallas import tpu_sc as plsc

N = 10000
D = 128
NUM_LAYERS = 3
ALPHA = 0.5

NC = 2
NS = 16
NT = NC * NS
L = 16

NP = 10240
NACC = 10112
CHUNK = 112
NCHUNK = 184
GB = 8
NBUF = 2
HC = CHUNK // 2
SLAB = NCHUNK * CHUNK
E_PAD = NS * SLAB

ROWS_PER_TILE = NACC // NS

_ISIG = np.empty((D,), np.int32)
for _k in range(D // 32):
    for _j in range(16):
        _ISIG[32 * _k + 2 * _j] = 32 * _k + _j
        _ISIG[32 * _k + 2 * _j + 1] = 32 * _k + 16 + _j

_mesh = plsc.VectorSubcoreMesh(
    core_axis_name="c", subcore_axis_name="s", num_cores=NC, num_subcores=NS
)

_sc_params = pltpu.CompilerParams()
if "needs_layout_passes" in pltpu.CompilerParams.__dataclass_fields__:
    _sc_params = dataclasses.replace(_sc_params, needs_layout_passes=False)
if "use_tc_tiling_on_sc" in pltpu.CompilerParams.__dataclass_fields__:
    _sc_params = dataclasses.replace(_sc_params, use_tc_tiling_on_sc=False)


@functools.partial(
    pl.kernel,
    out_type=jax.ShapeDtypeStruct((NT, 2, NP), jnp.float32),
    mesh=_mesh,
    scratch_types=[
        pltpu.VMEM((NCHUNK, CHUNK), jnp.int32),
        pltpu.VMEM((NCHUNK, CHUNK), jnp.int32),
        pltpu.VMEM((NCHUNK, CHUNK), jnp.float32),
        pltpu.VMEM((NP,), jnp.float32),
        pltpu.VMEM((NP,), jnp.float32),
    ],
    compiler_params=_sc_params,
)
def _deg_kernel(src_hbm, dst_hbm, ew_hbm, part_hbm, src_v, dst_v, ew_v,
                acco_v, acci_v):
    c = lax.axis_index("c")
    s = lax.axis_index("s")
    pltpu.sync_copy(src_hbm.at[s], src_v)
    pltpu.sync_copy(dst_hbm.at[s], dst_v)
    pltpu.sync_copy(ew_hbm.at[s], ew_v)

    zero = jnp.zeros((L,), jnp.float32)

    @pl.loop(0, NP // L)
    def _(i):
        acco_v.at[pl.ds(i * L, L)][...] = zero
        acci_v.at[pl.ds(i * L, L)][...] = zero

    half = NCHUNK // 2

    @pl.loop(0, half)
    def _(jj):
        j = c * half + jj

        @pl.loop(0, CHUNK // L)
        def _(g):
            sv = src_v.at[j, pl.ds(g * L, L)][...]
            dv = dst_v.at[j, pl.ds(g * L, L)][...]
            wv = ew_v.at[j, pl.ds(g * L, L)][...]
            plsc.addupdate_scatter(acco_v, [sv], wv)
            plsc.addupdate_scatter(acci_v, [dv], wv)

    w = c * NS + s
    pltpu.sync_copy(acco_v, part_hbm.at[w, 0])
    pltpu.sync_copy(acci_v, part_hbm.at[w, 1])


@functools.partial(
    pl.kernel,
    out_type=(
        jax.ShapeDtypeStruct((NP, D), jnp.float32),
        jax.ShapeDtypeStruct((NP, D), jnp.float32),
    ),
    mesh=_mesh,
    scratch_types=[
        pltpu.VMEM((2, GB, 2, HC), jnp.int32),
        pltpu.VMEM((2, GB, 2, HC), jnp.int32),
        pltpu.VMEM((2, GB, CHUNK), jnp.float32),
        pltpu.VMEM((CHUNK, D), jnp.bfloat16),
        pltpu.VMEM((CHUNK, D), jnp.bfloat16),
        pltpu.VMEM((CHUNK, D), jnp.float32),
        pltpu.VMEM((CHUNK, D), jnp.float32),
        pltpu.VMEM_SHARED((NACC, D), jnp.float32),
        pltpu.SemaphoreType.DMA,
        pltpu.SemaphoreType.DMA,
        pltpu.SemaphoreType.DMA,
        pltpu.SemaphoreType.DMA,
        pltpu.SemaphoreType.DMA,
        pltpu.SemaphoreType.DMA,
    ],
    compiler_params=_sc_params,
)
def _edge_kernel(u_hbm, v_hbm, src_hbm, dst_hbm, ew_hbm, seg1_hbm, seg2_hbm,
                 gidx_v, sidx_v, ew_v, in_a, in_b, out_a, out_b, acc_sh,
                 gsem_a, gsem_b, ssem_a, ssem_b, isem_a, isem_b):
    c = lax.axis_index("c")
    s = lax.axis_index("s")
    ins = (in_a, in_b)
    outs = (out_a, out_b)
    gsems = (gsem_a, gsem_b)
    ssems = (ssem_a, ssem_b)
    zero = jnp.zeros((L,), jnp.float32)
    hmask = jnp.int32(-65536)

    def start_gather(par, buf, j):
        for h in range(2):
            idx = gidx_v.at[par, j, h]
            dst = ins[buf].at[pl.ds(h * HC, HC)]

            @pl.when(c == 0)
            def _():
                pltpu.async_copy(u_hbm.at[idx], dst, gsems[buf])

            @pl.when(c != 0)
            def _():
                pltpu.async_copy(v_hbm.at[idx], dst, gsems[buf])

    def wait_gather(buf):
        for h in range(2):
            pltpu.make_async_copy(u_hbm.at[gidx_v.at[0, 0, 0]],
                                  ins[buf].at[pl.ds(h * HC, HC)],
                                  gsems[buf]).wait()

    def start_scatter(par, buf, j):
        for h in range(2):
            pltpu.async_copy(outs[buf].at[pl.ds(h * HC, HC)],
                             acc_sh.at[sidx_v.at[par, j, h]], ssems[buf],
                             add=True)

    def wait_scatter(buf):
        for h in range(2):
            pltpu.make_async_copy(outs[buf].at[pl.ds(h * HC, HC)],
                                  acc_sh.at[sidx_v.at[0, 0, 0]],
                                  ssems[buf]).wait()

    def start_stage(par, b):
        isem = (isem_a, isem_b)[par]
        blk = pl.ds(b * GB, GB)

        @pl.when(c == 0)
        def _():
            pltpu.async_copy(dst_hbm.at[s, blk], gidx_v.at[par], isem)
            pltpu.async_copy(src_hbm.at[s, blk], sidx_v.at[par], isem)

        @pl.when(c != 0)
        def _():
            pltpu.async_copy(src_hbm.at[s, blk], gidx_v.at[par], isem)
            pltpu.async_copy(dst_hbm.at[s, blk], sidx_v.at[par], isem)

        pltpu.async_copy(ew_hbm.at[s, blk], ew_v.at[par], isem)

    def wait_stage(par):
        isem = (isem_a, isem_b)[par]
        pltpu.make_async_copy(dst_hbm.at[s, pl.ds(0, GB)], gidx_v.at[par],
                              isem).wait()
        pltpu.make_async_copy(src_hbm.at[s, pl.ds(0, GB)], sidx_v.at[par],
                              isem).wait()
        pltpu.make_async_copy(ew_hbm.at[s, pl.ds(0, GB)], ew_v.at[par],
                              isem).wait()

    def scale(par, buf, j):
        rin = ins[buf]
        rout = outs[buf]

        @pl.loop(0, CHUNK // L)
        def _(g):
            wv = ew_v.at[par, j, pl.ds(g * L, L)][...]
            for i in range(L):
                w = lax.broadcast(wv[i], (L,))
                e = g * L + i
                for k in range(D // 32):
                    xb = rin.at[e, pl.ds(k * 32, 32)][...]
                    xi = plsc.bitcast(xb, jnp.int32)
                    lo = plsc.bitcast(xi << 16, jnp.float32)
                    hi = plsc.bitcast(xi & hmask, jnp.float32)
                    rout.at[e, pl.ds(32 * k, L)][...] = lo * w
                    rout.at[e, pl.ds(32 * k + L, L)][...] = hi * w

    @pl.loop(0, CHUNK)
    def _(e):
        for k in range(D // L):
            out_a.at[e, pl.ds(k * L, L)][...] = zero

    @pl.loop(0, ROWS_PER_TILE // CHUNK)
    def _(r):
        pltpu.sync_copy(
            out_a, acc_sh.at[pl.ds(s * ROWS_PER_TILE + r * CHUNK, CHUNK)])

    rem = ROWS_PER_TILE % CHUNK
    if rem:
        pltpu.sync_copy(
            out_a.at[pl.ds(0, rem)],
            acc_sh.at[pl.ds(s * ROWS_PER_TILE + ROWS_PER_TILE - rem, rem)])

    plsc.subcore_barrier()

    NB = NCHUNK // GB
    start_stage(0, 0)

    @pl.loop(0, NB)
    def _(b):
        par = b % 2
        first = b == 0

        @pl.when(par == 0)
        def _():
            wait_stage(0)

        @pl.when(par == 1)
        def _():
            wait_stage(1)

        @pl.when(b + 1 < NB)
        def _():
            @pl.when(par == 0)
            def _():
                start_stage(1, b + 1)

            @pl.when(par == 1)
            def _():
                start_stage(0, b + 1)

        for r in range(NBUF):
            @pl.when(jnp.logical_not(first))
            def _():
                wait_scatter(r)

            start_gather(par, r, r)

        @pl.loop(0, GB // NBUF)
        def _(t):
            for r in range(NBUF):
                j = NBUF * t + r
                wait_gather(r)

                @pl.when(t > 0)
                def _():
                    wait_scatter(r)

                scale(par, r, j)

                @pl.when(j + NBUF < GB)
                def _():
                    start_gather(par, r, j + NBUF)

                start_scatter(par, r, j)

    wait_scatter(0)
    wait_scatter(1)

    plsc.subcore_barrier()

    @pl.when(c == 0)
    def _():
        pltpu.sync_copy(acc_sh.at[pl.ds(s * ROWS_PER_TILE, ROWS_PER_TILE)],
                        seg1_hbm.at[pl.ds(s * ROWS_PER_TILE, ROWS_PER_TILE)])

    @pl.when(c != 0)
    def _():
        pltpu.sync_copy(acc_sh.at[pl.ds(s * ROWS_PER_TILE, ROWS_PER_TILE)],
                        seg2_hbm.at[pl.ds(s * ROWS_PER_TILE, ROWS_PER_TILE)])


_BL = 1280
_BR = 1024


def _degsum_body(part_ref, inv_ref):
    p = part_ref[...]
    deg = jnp.sum(p, axis=0)
    safe = jnp.where(deg > 0.0, deg, 1.0)
    inv_ref[...] = jnp.where(deg > 0.0, lax.rsqrt(safe), 0.0)


_degsum_call = pl.pallas_call(
    _degsum_body,
    grid=(NP // _BL,),
    in_specs=[pl.BlockSpec((NT, 2, _BL), lambda i: (0, 0, i))],
    out_specs=pl.BlockSpec((2, _BL), lambda i: (0, i)),
    out_shape=jax.ShapeDtypeStruct((2, NP), jnp.float32),
)


def _dot(a, b):
    return lax.dot_general(a, b, (((1,), (0,)), ((), ())),
                           precision=lax.Precision.HIGHEST,
                           preferred_element_type=jnp.float32)


def _uv_body(h_ref, w1_ref, w2_ref, cs_ref, u_ref, v_ref):
    h = h_ref[...]
    cs = cs_ref[...]
    u_ref[...] = (_dot(h, w1_ref[...]) * cs[:, 1:2]).astype(jnp.bfloat16)
    v_ref[...] = (_dot(h, w2_ref[...]) * cs[:, 0:1]).astype(jnp.bfloat16)


_uv_call = pl.pallas_call(
    _uv_body,
    grid=(NP // _BR,),
    in_specs=[
        pl.BlockSpec((_BR, D), lambda i: (i, 0)),
        pl.BlockSpec((D, D), lambda i: (0, 0)),
        pl.BlockSpec((D, D), lambda i: (0, 0)),
        pl.BlockSpec((_BR, 2), lambda i: (i, 0)),
    ],
    out_specs=[
        pl.BlockSpec((_BR, D), lambda i: (i, 0)),
        pl.BlockSpec((_BR, D), lambda i: (i, 0)),
    ],
    out_shape=[
        jax.ShapeDtypeStruct((NP, D), jnp.bfloat16),
        jax.ShapeDtypeStruct((NP, D), jnp.bfloat16),
    ],
)


def _layer_h(s1_ref, s2_ref, cs_ref, b1_ref, b2_ref):
    cs = cs_ref[...]
    t1 = cs[:, 0:1] * s1_ref[...] + b1_ref[...]
    t2 = cs[:, 1:2] * s2_ref[...] + b2_ref[...]
    return jnp.maximum(ALPHA * t1 + (1.0 - ALPHA) * t2, 0.0)


def _mid_body(s1_ref, s2_ref, cs_ref, b1_ref, b2_ref, jk_ref, w1_ref, w2_ref,
              jko_ref, u_ref, v_ref):
    h = _layer_h(s1_ref, s2_ref, cs_ref, b1_ref, b2_ref)
    cs = cs_ref[...]
    jko_ref[...] = jnp.maximum(jk_ref[...], h)
    u_ref[...] = (_dot(h, w1_ref[...]) * cs[:, 1:2]).astype(jnp.bfloat16)
    v_ref[...] = (_dot(h, w2_ref[...]) * cs[:, 0:1]).astype(jnp.bfloat16)


_mid_call = pl.pallas_call(
    _mid_body,
    grid=(NP // _BR,),
    in_specs=[
        pl.BlockSpec((_BR, D), lambda i: (i, 0)),
        pl.BlockSpec((_BR, D), lambda i: (i, 0)),
        pl.BlockSpec((_BR, 2), lambda i: (i, 0)),
        pl.BlockSpec((1, D), lambda i: (0, 0)),
        pl.BlockSpec((1, D), lambda i: (0, 0)),
        pl.BlockSpec((_BR, D), lambda i: (i, 0)),
        pl.BlockSpec((D, D), lambda i: (0, 0)),
        pl.BlockSpec((D, D), lambda i: (0, 0)),
    ],
    out_specs=[
        pl.BlockSpec((_BR, D), lambda i: (i, 0)),
        pl.BlockSpec((_BR, D), lambda i: (i, 0)),
        pl.BlockSpec((_BR, D), lambda i: (i, 0)),
    ],
    out_shape=[
        jax.ShapeDtypeStruct((NP, D), jnp.float32),
        jax.ShapeDtypeStruct((NP, D), jnp.bfloat16),
        jax.ShapeDtypeStruct((NP, D), jnp.bfloat16),
    ],
)


def _fin_body(jk_ref, wl_ref, bl_ref, out_ref):
    out_ref[...] = _dot(jk_ref[...], wl_ref[...]) + bl_ref[...]


_fin_call = pl.pallas_call(
    _fin_body,
    grid=(NP // _BR,),
    in_specs=[
        pl.BlockSpec((_BR, D), lambda i: (i, 0)),
        pl.BlockSpec((D, D), lambda i: (0, 0)),
        pl.BlockSpec((1, D), lambda i: (0, 0)),
    ],
    out_specs=pl.BlockSpec((_BR, D), lambda i: (i, 0)),
    out_shape=jax.ShapeDtypeStruct((NP, D), jnp.float32),
)


@jax.jit
def kernel(x, edge_index, edge_weight, W_s2d, b_s2d, W_d2s, b_d2s, W_lin,
           b_lin):
    E = edge_index.shape[1]
    pad = E_PAD - E

    src = jnp.concatenate([edge_index[0], jnp.zeros((pad,), jnp.int32)])
    dst = jnp.concatenate([edge_index[1], jnp.zeros((pad,), jnp.int32)])
    ew = jnp.concatenate([edge_weight, jnp.zeros((pad,), jnp.float32)])
    src3 = src.reshape(NS, NCHUNK, CHUNK)
    dst3 = dst.reshape(NS, NCHUNK, CHUNK)
    ew3 = ew.reshape(NS, NCHUNK, CHUNK)
    src4 = src.reshape(NS, NCHUNK, 2, HC)
    dst4 = dst.reshape(NS, NCHUNK, 2, HC)

    xp = jnp.zeros((NP, D), jnp.float32).at[:N].set(x)

    part = _deg_kernel(src3, dst3, ew3)
    inv = _degsum_call(part)
    colscale = inv.T

    b1 = b_s2d.reshape(NUM_LAYERS, 1, D)
    b2 = b_d2s.reshape(NUM_LAYERS, 1, D)

    isig = jnp.asarray(_ISIG)
    W1p = W_s2d[:, :, isig]
    W2p = W_d2s[:, :, isig]

    u, v = _uv_call(xp, W1p[0], W2p[0], colscale)
    jk = jnp.zeros((NP, D), jnp.float32)

    zw = jnp.zeros((1, D, D), jnp.float32)
    W1n = jnp.concatenate([W1p[1:], zw])
    W2n = jnp.concatenate([W2p[1:], zw])

    def body(carry, xs):
        u, v, jk = carry
        w1n, w2n, b1i, b2i = xs
        seg1, seg2 = _edge_kernel(u, v, src4, dst4, ew3)
        jk, u, v = _mid_call(seg1, seg2, colscale, b1i, b2i, jk, w1n, w2n)
        return (u, v, jk), None

    (u, v, jk), _ = lax.scan(body, (u, v, jk), (W1n, W2n, b1, b2))
    out = _fin_call(jk, W_lin, b_lin.reshape(1, D))
    return out[:N]
